# baseline (speedup 1.0000x reference)
import functools

import jax
import jax.numpy as jnp
from jax import lax
from jax.experimental import pallas as pl
from jax.experimental.pallas import tpu as pltpu
from jax.experimental.pallas import tpu_sc as plsc

N = 10000
E = 160000
S = 256
O = 128
L = 3
T = 3

NC = 2
NS = 16
CH = 128
NCHUNK = 80
EPAD = NS * CH * NCHUNK
NPAD = ((N + NS * 8 - 1) // (NS * 8)) * (NS * 8)


def _segsum_sc(s2n, srcs4, dst3, zeros):
    mesh = plsc.VectorSubcoreMesh(core_axis_name="c", subcore_axis_name="s")
    zrows = NPAD // NS
    wrows = NPAD // NS
    K = 16
    NG = NCHUNK // K
    NB = 2
    A = NB - 1

    @functools.partial(
        pl.kernel,
        out_type=jax.ShapeDtypeStruct((NC, NPAD, 128), jnp.float32),
        mesh=mesh,
        scratch_types=[
            pltpu.VMEM((2, K, CH), jnp.int32),
            pltpu.VMEM((2, K, CH), jnp.int32),
            pltpu.VMEM((NB, CH, 128), jnp.float32),
            pltpu.VMEM_SHARED((NPAD, 128), jnp.float32),
            pltpu.SemaphoreType.DMA,
            pltpu.SemaphoreType.DMA,
            pltpu.SemaphoreType.DMA,
            pltpu.SemaphoreType.DMA,
            pltpu.SemaphoreType.DMA,
            pltpu.SemaphoreType.DMA,
            pltpu.SemaphoreType.DMA,
            pltpu.SemaphoreType.DMA,
            pltpu.SemaphoreType.DMA,
        ],
    )
    def seg_kernel(s_hbm, src_hbm, dst_hbm, zero_hbm, out_hbm,
                   sidx, didx, rows, agg,
                   g0, g1, g2, g3, s0, s1, s2, s3, semi):
        c = lax.axis_index("c")
        t = lax.axis_index("s")
        gsems = (g0, g1, g2, g3)
        ssems = (s0, s1, s2, s3)

        def gather(p, g, b, sem):
            pltpu.async_copy(s_hbm.at[sidx.at[p, g]], rows.at[b], sem)

        def gather_wait(b, sem):
            pltpu.make_async_copy(s_hbm.at[sidx.at[0, 0]], rows.at[b],
                                  sem).wait()

        def scat(p, g, b, sem):
            pltpu.async_copy(rows.at[b], agg.at[didx.at[p, g]], sem,
                             add=True)

        def scat_wait(b, sem):
            pltpu.make_async_copy(rows.at[b], agg.at[didx.at[0, 0]],
                                  sem).wait()

        pltpu.sync_copy(src_hbm.at[c, t, pl.ds(0, K)], sidx.at[0])
        pltpu.sync_copy(dst_hbm.at[t, pl.ds(0, K)], didx.at[0])
        pltpu.sync_copy(zero_hbm.at[pl.ds(t * zrows, zrows)],
                        agg.at[pl.ds(t * zrows, zrows)])
        plsc.subcore_barrier()

        for ig in range(NG):
            p = ig % 2
            if ig + 1 < NG:
                pltpu.async_copy(src_hbm.at[c, t, pl.ds((ig + 1) * K, K)],
                                 sidx.at[1 - p], semi)
                pltpu.async_copy(dst_hbm.at[t, pl.ds((ig + 1) * K, K)],
                                 didx.at[1 - p], semi)

            for b in range(A):
                gather(p, b, b, gsems[b])
            for g in range(NB - A):
                bA = g + A
                gather(p, g + A, bA, gsems[bA])
                gather_wait(g, gsems[g])
                scat(p, g, g, ssems[g])

            def rnd(r, carry, p=p):
                gbase = (NB - A) + NB * r
                for b in range(NB):
                    scat_wait(b, ssems[b])
                    gather(p, gbase + b + A, b, gsems[b])
                    bg = (b + NB - A) % NB
                    gather_wait(bg, gsems[bg])
                    scat(p, gbase + b, bg, ssems[bg])
                return carry

            lax.fori_loop(0, (K - NB) // NB, rnd, 0)

            for i in range(A):
                g = K - A + i
                b = g % NB
                gather_wait(b, gsems[b])
                scat(p, g, b, ssems[b])
            for b in range(NB):
                scat_wait(b, ssems[b])

            if ig + 1 < NG:
                pltpu.make_async_copy(src_hbm.at[c, t, pl.ds(0, K)],
                                      sidx.at[1 - p], semi).wait()
                pltpu.make_async_copy(dst_hbm.at[t, pl.ds(0, K)],
                                      didx.at[1 - p], semi).wait()

        plsc.subcore_barrier()
        pltpu.sync_copy(agg.at[pl.ds(t * wrows, wrows)],
                        out_hbm.at[c, pl.ds(t * wrows, wrows)])

    return seg_kernel(s2n, srcs4, dst3, zeros)


def _tc_mm(a_ws, bias, c_arr, want_pre, want_act, want_stk, rb=1000):
    n = a_ws[0][0].shape[0]
    m = a_ws[0][1].shape[1]
    grid = (n // rb,)
    in_specs = []
    operands = []
    for (a, w) in a_ws:
        k = a.shape[1]
        in_specs.append(pl.BlockSpec((rb, k), lambda i: (i, 0)))
        in_specs.append(pl.BlockSpec((k, m), lambda i: (0, 0)))
        operands += [a, w]
    if bias is not None:
        in_specs.append(pl.BlockSpec((1, m), lambda i: (0, 0)))
        operands.append(bias.reshape(1, m))
    if c_arr is not None:
        in_specs.append(pl.BlockSpec((rb, m), lambda i: (i, 0)))
        operands.append(c_arr)

    out_shapes, out_specs = [], []
    if want_pre:
        out_shapes.append(jax.ShapeDtypeStruct((n, m), jnp.float32))
        out_specs.append(pl.BlockSpec((rb, m), lambda i: (i, 0)))
    if want_act:
        out_shapes.append(jax.ShapeDtypeStruct((n, m), jnp.float32))
        out_specs.append(pl.BlockSpec((rb, m), lambda i: (i, 0)))
    if want_stk:
        out_shapes.append(jax.ShapeDtypeStruct((2, n, m // 2), jnp.float32))
        out_specs.append(pl.BlockSpec((2, rb, m // 2), lambda i: (0, i, 0)))

    n_a = len(a_ws)
    has_b = bias is not None
    has_c = c_arr is not None

    def body(*refs):
        pos = 2 * n_a
        acc = None
        for j in range(n_a):
            prod = jnp.dot(refs[2 * j][...], refs[2 * j + 1][...],
                           preferred_element_type=jnp.float32)
            acc = prod if acc is None else acc + prod
        if has_b:
            acc = acc + refs[pos][...]
            pos += 1
        if has_c:
            acc = acc + refs[pos][...]
            pos += 1
        outs = refs[pos:]
        oi = 0
        if want_pre:
            outs[oi][...] = acc
            oi += 1
        if want_act or want_stk:
            act = jnp.tanh(acc)
        if want_act:
            outs[oi][...] = act
            oi += 1
        if want_stk:
            outs[oi][0] = act[:, : m // 2]
            outs[oi][1] = act[:, m // 2:]

    res = pl.pallas_call(
        body,
        grid=grid,
        in_specs=in_specs,
        out_specs=out_specs if len(out_specs) > 1 else out_specs[0],
        out_shape=out_shapes if len(out_shapes) > 1 else out_shapes[0],
    )(*operands)
    return res if isinstance(res, (list, tuple)) else (res,)


def kernel(x, edge_index, W_s0, b_s0, W_o0, b_o0, W_s1, b_s1, W_o1, b_o1,
           W_s2, b_s2, W_o2, b_o2):
    src = edge_index[0]
    dst = edge_index[1]
    src_pad = jnp.concatenate(
        [src, jnp.zeros((EPAD - E,), jnp.int32)])
    srcs4 = jnp.stack([src_pad, src_pad + N]).reshape(2, NS, NCHUNK, CH)
    dst3 = jnp.concatenate(
        [dst, jnp.full((EPAD - E,), N, jnp.int32)]).reshape(NS, NCHUNK, CH)
    zeros = jnp.zeros((NPAD, 128), jnp.float32)

    params = [(W_s0, b_s0, W_o0, b_o0), (W_s1, b_s1, W_o1, b_o1),
              (W_s2, b_s2, W_o2, b_o2)]
    dims = [256, 640, 1024]
    h = x
    out = None
    for l in range(L):
        W_s, b_s, W_o, b_o = params[l]
        d = dims[l]
        W_h, W_a = W_s[:d], W_s[d:]
        hWb, s, s_stk = _tc_mm([(h, W_h)], b_s, None, True, True, True)
        for t in range(1, T):
            agg = _segsum_sc(s_stk.reshape(2 * N, 128), srcs4, dst3, zeros)
            agg = agg[:, :N]
            need_stk = t < T - 1
            res = _tc_mm([(agg[0], W_a[:128]), (agg[1], W_a[128:])],
                         None, hWb, False, True, need_stk)
            s = res[0]
            if need_stk:
                s_stk = res[1]
        (out,) = _tc_mm([(h, W_o[:d]), (s, W_o[d:])], b_o, None,
                        True, False, False)
        if l < L - 1:
            h = jnp.concatenate([h, s, out], axis=1)
    return out


# speedup vs baseline: 1.0023x; 1.0023x over previous
import functools

import jax
import jax.numpy as jnp
from jax import lax
from jax.experimental import pallas as pl
from jax.experimental.pallas import tpu as pltpu
from jax.experimental.pallas import tpu_sc as plsc

N = 10000
E = 160000
S = 256
O = 128
L = 3
T = 3

NC = 2
NS = 16
CH = 32
NCHUNK = 320
EPAD = NS * CH * NCHUNK
NPAD = ((N + NS * 8 - 1) // (NS * 8)) * (NS * 8)


def _segsum_sc(s2n, srcs4, dst3, zeros):
    mesh = plsc.VectorSubcoreMesh(core_axis_name="c", subcore_axis_name="s")
    zrows = NPAD // NS
    wrows = NPAD // NS
    K = 32
    NG = NCHUNK // K
    NB = 8
    A = NB - 1

    @functools.partial(
        pl.kernel,
        out_type=jax.ShapeDtypeStruct((NC, NPAD, 128), jnp.float32),
        mesh=mesh,
        scratch_types=[
            pltpu.VMEM((2, K, CH), jnp.int32),
            pltpu.VMEM((2, K, CH), jnp.int32),
            pltpu.VMEM((NB, CH, 128), jnp.float32),
            pltpu.VMEM_SHARED((NPAD, 128), jnp.float32),
        ] + [pltpu.SemaphoreType.DMA] * (2 * NB + 1),
    )
    def seg_kernel(s_hbm, src_hbm, dst_hbm, zero_hbm, out_hbm,
                   sidx, didx, rows, agg, *sems):
        semi = sems[2 * NB]
        c = lax.axis_index("c")
        t = lax.axis_index("s")
        gsems = sems[:NB]
        ssems = sems[NB:2 * NB]

        def gather(p, g, b, sem):
            pltpu.async_copy(s_hbm.at[sidx.at[p, g]], rows.at[b], sem)

        def gather_wait(b, sem):
            pltpu.make_async_copy(s_hbm.at[sidx.at[0, 0]], rows.at[b],
                                  sem).wait()

        def scat(p, g, b, sem):
            pltpu.async_copy(rows.at[b], agg.at[didx.at[p, g]], sem,
                             add=True)

        def scat_wait(b, sem):
            pltpu.make_async_copy(rows.at[b], agg.at[didx.at[0, 0]],
                                  sem).wait()

        pltpu.sync_copy(src_hbm.at[c, t, pl.ds(0, K)], sidx.at[0])
        pltpu.sync_copy(dst_hbm.at[t, pl.ds(0, K)], didx.at[0])
        pltpu.sync_copy(zero_hbm.at[pl.ds(t * zrows, zrows)],
                        agg.at[pl.ds(t * zrows, zrows)])
        plsc.subcore_barrier()

        for ig in range(NG):
            p = ig % 2
            if ig + 1 < NG:
                pltpu.async_copy(src_hbm.at[c, t, pl.ds((ig + 1) * K, K)],
                                 sidx.at[1 - p], semi)
                pltpu.async_copy(dst_hbm.at[t, pl.ds((ig + 1) * K, K)],
                                 didx.at[1 - p], semi)

            for b in range(A):
                gather(p, b, b, gsems[b])
            for g in range(NB - A):
                bA = g + A
                gather(p, g + A, bA, gsems[bA])
                gather_wait(g, gsems[g])
                scat(p, g, g, ssems[g])

            def rnd(r, carry, p=p):
                gbase = (NB - A) + NB * r
                for b in range(NB):
                    scat_wait(b, ssems[b])
                    gather(p, gbase + b + A, b, gsems[b])
                    bg = (b + NB - A) % NB
                    gather_wait(bg, gsems[bg])
                    scat(p, gbase + b, bg, ssems[bg])
                return carry

            lax.fori_loop(0, (K - NB) // NB, rnd, 0)

            for i in range(A):
                g = K - A + i
                b = g % NB
                gather_wait(b, gsems[b])
                scat(p, g, b, ssems[b])
            for b in range(NB):
                scat_wait(b, ssems[b])

            if ig + 1 < NG:
                pltpu.make_async_copy(src_hbm.at[c, t, pl.ds(0, K)],
                                      sidx.at[1 - p], semi).wait()
                pltpu.make_async_copy(dst_hbm.at[t, pl.ds(0, K)],
                                      didx.at[1 - p], semi).wait()

        plsc.subcore_barrier()
        pltpu.sync_copy(agg.at[pl.ds(t * wrows, wrows)],
                        out_hbm.at[c, pl.ds(t * wrows, wrows)])

    return seg_kernel(s2n, srcs4, dst3, zeros)


def _tc_mm(a_ws, bias, c_arr, want_pre, want_act, want_stk, rb=1000):
    n = a_ws[0][0].shape[0]
    m = a_ws[0][1].shape[1]
    grid = (n // rb,)
    in_specs = []
    operands = []
    for (a, w) in a_ws:
        k = a.shape[1]
        in_specs.append(pl.BlockSpec((rb, k), lambda i: (i, 0)))
        in_specs.append(pl.BlockSpec((k, m), lambda i: (0, 0)))
        operands += [a, w]
    if bias is not None:
        in_specs.append(pl.BlockSpec((1, m), lambda i: (0, 0)))
        operands.append(bias.reshape(1, m))
    if c_arr is not None:
        in_specs.append(pl.BlockSpec((rb, m), lambda i: (i, 0)))
        operands.append(c_arr)

    out_shapes, out_specs = [], []
    if want_pre:
        out_shapes.append(jax.ShapeDtypeStruct((n, m), jnp.float32))
        out_specs.append(pl.BlockSpec((rb, m), lambda i: (i, 0)))
    if want_act:
        out_shapes.append(jax.ShapeDtypeStruct((n, m), jnp.float32))
        out_specs.append(pl.BlockSpec((rb, m), lambda i: (i, 0)))
    if want_stk:
        out_shapes.append(jax.ShapeDtypeStruct((2, n, m // 2), jnp.float32))
        out_specs.append(pl.BlockSpec((2, rb, m // 2), lambda i: (0, i, 0)))

    n_a = len(a_ws)
    has_b = bias is not None
    has_c = c_arr is not None

    def body(*refs):
        pos = 2 * n_a
        acc = None
        for j in range(n_a):
            prod = jnp.dot(refs[2 * j][...], refs[2 * j + 1][...],
                           preferred_element_type=jnp.float32)
            acc = prod if acc is None else acc + prod
        if has_b:
            acc = acc + refs[pos][...]
            pos += 1
        if has_c:
            acc = acc + refs[pos][...]
            pos += 1
        outs = refs[pos:]
        oi = 0
        if want_pre:
            outs[oi][...] = acc
            oi += 1
        if want_act or want_stk:
            act = jnp.tanh(acc)
        if want_act:
            outs[oi][...] = act
            oi += 1
        if want_stk:
            outs[oi][0] = act[:, : m // 2]
            outs[oi][1] = act[:, m // 2:]

    res = pl.pallas_call(
        body,
        grid=grid,
        in_specs=in_specs,
        out_specs=out_specs if len(out_specs) > 1 else out_specs[0],
        out_shape=out_shapes if len(out_shapes) > 1 else out_shapes[0],
    )(*operands)
    return res if isinstance(res, (list, tuple)) else (res,)


def kernel(x, edge_index, W_s0, b_s0, W_o0, b_o0, W_s1, b_s1, W_o1, b_o1,
           W_s2, b_s2, W_o2, b_o2):
    src = edge_index[0]
    dst = edge_index[1]
    src_pad = jnp.concatenate(
        [src, jnp.zeros((EPAD - E,), jnp.int32)])
    srcs4 = jnp.stack([src_pad, src_pad + N]).reshape(2, NS, NCHUNK, CH)
    dst3 = jnp.concatenate(
        [dst, jnp.full((EPAD - E,), N, jnp.int32)]).reshape(NS, NCHUNK, CH)
    zeros = jnp.zeros((NPAD, 128), jnp.float32)

    params = [(W_s0, b_s0, W_o0, b_o0), (W_s1, b_s1, W_o1, b_o1),
              (W_s2, b_s2, W_o2, b_o2)]
    dims = [256, 640, 1024]
    h = x
    out = None
    for l in range(L):
        W_s, b_s, W_o, b_o = params[l]
        d = dims[l]
        W_h, W_a = W_s[:d], W_s[d:]
        hWb, s, s_stk = _tc_mm([(h, W_h)], b_s, None, True, True, True)
        for t in range(1, T):
            agg = _segsum_sc(s_stk.reshape(2 * N, 128), srcs4, dst3, zeros)
            agg = agg[:, :N]
            need_stk = t < T - 1
            res = _tc_mm([(agg[0], W_a[:128]), (agg[1], W_a[128:])],
                         None, hWb, False, True, need_stk)
            s = res[0]
            if need_stk:
                s_stk = res[1]
        (out,) = _tc_mm([(h, W_o[:d]), (s, W_o[d:])], b_o, None,
                        True, False, False)
        if l < L - 1:
            h = jnp.concatenate([h, s, out], axis=1)
    return out


# speedup vs baseline: 1.0266x; 1.0243x over previous
import functools

import jax
import jax.numpy as jnp
from jax import lax
from jax.experimental import pallas as pl
from jax.experimental.pallas import tpu as pltpu
from jax.experimental.pallas import tpu_sc as plsc

N = 10000
E = 160000
S = 256
O = 128
L = 3
T = 3

NC = 2
NS = 16
CH = 64
NCHUNK = 160
EPAD = NS * CH * NCHUNK
NPAD = ((N + NS * 8 - 1) // (NS * 8)) * (NS * 8)


def _segsum_sc(s2n, srcs4, dst3, zeros):
    mesh = plsc.VectorSubcoreMesh(core_axis_name="c", subcore_axis_name="s")
    zrows = NPAD // NS
    wrows = NPAD // NS
    K = 32
    NG = NCHUNK // K
    NB = 4
    A = NB - 1

    @functools.partial(
        pl.kernel,
        out_type=jax.ShapeDtypeStruct((NC, NPAD, 128), jnp.float32),
        mesh=mesh,
        scratch_types=[
            pltpu.VMEM((2, K, CH), jnp.int32),
            pltpu.VMEM((2, K, CH), jnp.int32),
            pltpu.VMEM((NB, CH, 128), jnp.float32),
            pltpu.VMEM_SHARED((NPAD, 128), jnp.float32),
            pltpu.SemaphoreType.DMA,
            pltpu.SemaphoreType.DMA,
            pltpu.SemaphoreType.DMA,
            pltpu.SemaphoreType.DMA,
            pltpu.SemaphoreType.DMA,
            pltpu.SemaphoreType.DMA,
            pltpu.SemaphoreType.DMA,
            pltpu.SemaphoreType.DMA,
            pltpu.SemaphoreType.DMA,
        ],
    )
    def seg_kernel(s_hbm, src_hbm, dst_hbm, zero_hbm, out_hbm,
                   sidx, didx, rows, agg,
                   g0, g1, g2, g3, s0, s1, s2, s3, semi):
        c = lax.axis_index("c")
        t = lax.axis_index("s")
        gsems = (g0, g1, g2, g3)
        ssems = (s0, s1, s2, s3)

        def gather(p, g, b, sem):
            pltpu.async_copy(s_hbm.at[sidx.at[p, g]], rows.at[b], sem)

        def gather_wait(b, sem):
            pltpu.make_async_copy(s_hbm.at[sidx.at[0, 0]], rows.at[b],
                                  sem).wait()

        def scat(p, g, b, sem):
            pltpu.async_copy(rows.at[b], agg.at[didx.at[p, g]], sem,
                             add=True)

        def scat_wait(b, sem):
            pltpu.make_async_copy(rows.at[b], agg.at[didx.at[0, 0]],
                                  sem).wait()

        pltpu.sync_copy(src_hbm.at[c, t, pl.ds(0, K)], sidx.at[0])
        pltpu.sync_copy(dst_hbm.at[t, pl.ds(0, K)], didx.at[0])
        pltpu.sync_copy(zero_hbm.at[pl.ds(t * zrows, zrows)],
                        agg.at[pl.ds(t * zrows, zrows)])
        plsc.subcore_barrier()

        for ig in range(NG):
            p = ig % 2
            if ig + 1 < NG:
                pltpu.async_copy(src_hbm.at[c, t, pl.ds((ig + 1) * K, K)],
                                 sidx.at[1 - p], semi)
                pltpu.async_copy(dst_hbm.at[t, pl.ds((ig + 1) * K, K)],
                                 didx.at[1 - p], semi)

            for b in range(A):
                gather(p, b, b, gsems[b])
            for g in range(NB - A):
                bA = g + A
                gather(p, g + A, bA, gsems[bA])
                gather_wait(g, gsems[g])
                scat(p, g, g, ssems[g])

            def rnd(r, carry, p=p):
                gbase = (NB - A) + NB * r
                for b in range(NB):
                    scat_wait(b, ssems[b])
                    gather(p, gbase + b + A, b, gsems[b])
                    bg = (b + NB - A) % NB
                    gather_wait(bg, gsems[bg])
                    scat(p, gbase + b, bg, ssems[bg])
                return carry

            lax.fori_loop(0, (K - NB) // NB, rnd, 0)

            for i in range(A):
                g = K - A + i
                b = g % NB
                gather_wait(b, gsems[b])
                scat(p, g, b, ssems[b])
            for b in range(NB):
                scat_wait(b, ssems[b])

            if ig + 1 < NG:
                pltpu.make_async_copy(src_hbm.at[c, t, pl.ds(0, K)],
                                      sidx.at[1 - p], semi).wait()
                pltpu.make_async_copy(dst_hbm.at[t, pl.ds(0, K)],
                                      didx.at[1 - p], semi).wait()

        plsc.subcore_barrier()
        pltpu.sync_copy(agg.at[pl.ds(t * wrows, wrows)],
                        out_hbm.at[c, pl.ds(t * wrows, wrows)])

    return seg_kernel(s2n, srcs4, dst3, zeros)


def _tc_mm(a_ws, bias, c_arr, want_pre, want_act, want_stk, rb=1000):
    n = a_ws[0][0].shape[0]
    m = a_ws[0][1].shape[1]
    grid = (n // rb,)
    in_specs = []
    operands = []
    for (a, w) in a_ws:
        k = a.shape[1]
        in_specs.append(pl.BlockSpec((rb, k), lambda i: (i, 0)))
        in_specs.append(pl.BlockSpec((k, m), lambda i: (0, 0)))
        operands += [a, w]
    if bias is not None:
        in_specs.append(pl.BlockSpec((1, m), lambda i: (0, 0)))
        operands.append(bias.reshape(1, m))
    if c_arr is not None:
        in_specs.append(pl.BlockSpec((rb, m), lambda i: (i, 0)))
        operands.append(c_arr)

    out_shapes, out_specs = [], []
    if want_pre:
        out_shapes.append(jax.ShapeDtypeStruct((n, m), jnp.float32))
        out_specs.append(pl.BlockSpec((rb, m), lambda i: (i, 0)))
    if want_act:
        out_shapes.append(jax.ShapeDtypeStruct((n, m), jnp.float32))
        out_specs.append(pl.BlockSpec((rb, m), lambda i: (i, 0)))
    if want_stk:
        out_shapes.append(jax.ShapeDtypeStruct((2, n, m // 2), jnp.float32))
        out_specs.append(pl.BlockSpec((2, rb, m // 2), lambda i: (0, i, 0)))

    n_a = len(a_ws)
    has_b = bias is not None
    has_c = c_arr is not None

    def body(*refs):
        pos = 2 * n_a
        acc = None
        for j in range(n_a):
            prod = jnp.dot(refs[2 * j][...], refs[2 * j + 1][...],
                           preferred_element_type=jnp.float32)
            acc = prod if acc is None else acc + prod
        if has_b:
            acc = acc + refs[pos][...]
            pos += 1
        if has_c:
            acc = acc + refs[pos][...]
            pos += 1
        outs = refs[pos:]
        oi = 0
        if want_pre:
            outs[oi][...] = acc
            oi += 1
        if want_act or want_stk:
            act = jnp.tanh(acc)
        if want_act:
            outs[oi][...] = act
            oi += 1
        if want_stk:
            outs[oi][0] = act[:, : m // 2]
            outs[oi][1] = act[:, m // 2:]

    res = pl.pallas_call(
        body,
        grid=grid,
        in_specs=in_specs,
        out_specs=out_specs if len(out_specs) > 1 else out_specs[0],
        out_shape=out_shapes if len(out_shapes) > 1 else out_shapes[0],
    )(*operands)
    return res if isinstance(res, (list, tuple)) else (res,)


def kernel(x, edge_index, W_s0, b_s0, W_o0, b_o0, W_s1, b_s1, W_o1, b_o1,
           W_s2, b_s2, W_o2, b_o2):
    src = edge_index[0]
    dst = edge_index[1]
    src_pad = jnp.concatenate(
        [src, jnp.zeros((EPAD - E,), jnp.int32)])
    srcs4 = jnp.stack([src_pad, src_pad + N]).reshape(2, NS, NCHUNK, CH)
    dst3 = jnp.concatenate(
        [dst, jnp.full((EPAD - E,), N, jnp.int32)]).reshape(NS, NCHUNK, CH)
    zeros = jnp.zeros((NPAD, 128), jnp.float32)

    params = [(W_s0, b_s0, W_o0, b_o0), (W_s1, b_s1, W_o1, b_o1),
              (W_s2, b_s2, W_o2, b_o2)]
    dims = [256, 640, 1024]
    h = x
    out = None
    for l in range(L):
        W_s, b_s, W_o, b_o = params[l]
        d = dims[l]
        W_h, W_a = W_s[:d], W_s[d:]
        hWb, s, s_stk = _tc_mm([(h, W_h)], b_s, None, True, True, True)
        for t in range(1, T):
            agg = _segsum_sc(s_stk.reshape(2 * N, 128), srcs4, dst3, zeros)
            agg = agg[:, :N]
            need_stk = t < T - 1
            res = _tc_mm([(agg[0], W_a[:128]), (agg[1], W_a[128:])],
                         None, hWb, False, True, need_stk)
            s = res[0]
            if need_stk:
                s_stk = res[1]
        (out,) = _tc_mm([(h, W_o[:d]), (s, W_o[d:])], b_o, None,
                        True, False, False)
        if l < L - 1:
            h = jnp.concatenate([h, s, out], axis=1)
    return out


# speedup vs baseline: 1.0689x; 1.0412x over previous
import functools

import jax
import jax.numpy as jnp
from jax import lax
from jax.experimental import pallas as pl
from jax.experimental.pallas import tpu as pltpu
from jax.experimental.pallas import tpu_sc as plsc

N = 10000
E = 160000
S = 256
O = 128
L = 3
T = 3

NC = 2
NS = 16
CH = 64
NCHUNK = 160
EPAD = NS * CH * NCHUNK
NP = 10240
RB = 2048


def _segsum_sc(s2n, srcs4, dst3, zeros):
    mesh = plsc.VectorSubcoreMesh(core_axis_name="c", subcore_axis_name="s")
    zrows = NP // NS
    wrows = NP // NS
    K = 16
    NG = NCHUNK // K
    NB = 4
    A = NB - 1

    @functools.partial(
        pl.kernel,
        out_type=jax.ShapeDtypeStruct((NC, NP, 128), jnp.float32),
        mesh=mesh,
        scratch_types=[
            pltpu.VMEM((2, K, CH), jnp.int32),
            pltpu.VMEM((2, K, CH), jnp.int32),
            pltpu.VMEM((NB, CH, 128), jnp.float32),
            pltpu.VMEM_SHARED((NP, 128), jnp.float32),
            pltpu.SemaphoreType.DMA,
            pltpu.SemaphoreType.DMA,
            pltpu.SemaphoreType.DMA,
            pltpu.SemaphoreType.DMA,
            pltpu.SemaphoreType.DMA,
            pltpu.SemaphoreType.DMA,
            pltpu.SemaphoreType.DMA,
            pltpu.SemaphoreType.DMA,
            pltpu.SemaphoreType.DMA,
        ],
    )
    def seg_kernel(s_hbm, src_hbm, dst_hbm, zero_hbm, out_hbm,
                   sidx, didx, rows, agg,
                   g0, g1, g2, g3, s0, s1, s2, s3, semi):
        c = lax.axis_index("c")
        t = lax.axis_index("s")
        gsems = (g0, g1, g2, g3)
        ssems = (s0, s1, s2, s3)

        def gather(p, g, b, sem):
            pltpu.async_copy(s_hbm.at[sidx.at[p, g]], rows.at[b], sem)

        def gather_wait(b, sem):
            pltpu.make_async_copy(s_hbm.at[sidx.at[0, 0]], rows.at[b],
                                  sem).wait()

        def scat(p, g, b, sem):
            pltpu.async_copy(rows.at[b], agg.at[didx.at[p, g]], sem,
                             add=True)

        def scat_wait(b, sem):
            pltpu.make_async_copy(rows.at[b], agg.at[didx.at[0, 0]],
                                  sem).wait()

        pltpu.sync_copy(src_hbm.at[c, t, pl.ds(0, K)], sidx.at[0])
        pltpu.sync_copy(dst_hbm.at[t, pl.ds(0, K)], didx.at[0])
        pltpu.sync_copy(zero_hbm.at[pl.ds(t * zrows, zrows)],
                        agg.at[pl.ds(t * zrows, zrows)])
        plsc.subcore_barrier()

        for ig in range(NG):
            p = ig % 2
            if ig + 1 < NG:
                pltpu.async_copy(src_hbm.at[c, t, pl.ds((ig + 1) * K, K)],
                                 sidx.at[1 - p], semi)
                pltpu.async_copy(dst_hbm.at[t, pl.ds((ig + 1) * K, K)],
                                 didx.at[1 - p], semi)

            for b in range(A):
                gather(p, b, b, gsems[b])
            for g in range(NB - A):
                bA = g + A
                gather(p, g + A, bA, gsems[bA])
                gather_wait(g, gsems[g])
                scat(p, g, g, ssems[g])

            def rnd(r, carry, p=p):
                gbase = (NB - A) + NB * r
                for b in range(NB):
                    scat_wait(b, ssems[b])
                    gather(p, gbase + b + A, b, gsems[b])
                    bg = (b + NB - A) % NB
                    gather_wait(bg, gsems[bg])
                    scat(p, gbase + b, bg, ssems[bg])
                return carry

            lax.fori_loop(0, (K - NB) // NB, rnd, 0)

            for i in range(A):
                g = K - A + i
                b = g % NB
                gather_wait(b, gsems[b])
                scat(p, g, b, ssems[b])
            for b in range(NB):
                scat_wait(b, ssems[b])

            if ig + 1 < NG:
                pltpu.make_async_copy(src_hbm.at[c, t, pl.ds(0, K)],
                                      sidx.at[1 - p], semi).wait()
                pltpu.make_async_copy(dst_hbm.at[t, pl.ds(0, K)],
                                      didx.at[1 - p], semi).wait()

        plsc.subcore_barrier()
        pltpu.sync_copy(agg.at[pl.ds(t * wrows, wrows)],
                        out_hbm.at[c, pl.ds(t * wrows, wrows)])

    return seg_kernel(s2n, srcs4, dst3, zeros)


def _tc_mm(a_ws, bias, c_arr, agg_w, want_pre, want_act, want_stk,
           out_ws=None, out_bias=None, act_w=None):
    m = a_ws[0][1].shape[1] if a_ws else agg_w[1].shape[1]
    grid = (NP // RB,)
    in_specs = []
    operands = []
    for (a, w) in a_ws:
        k = a.shape[1]
        in_specs.append(pl.BlockSpec((RB, k), lambda i: (i, 0)))
        in_specs.append(pl.BlockSpec((k, m), lambda i: (0, 0)))
        operands += [a, w]
    if bias is not None:
        in_specs.append(pl.BlockSpec((1, m), lambda i: (0, 0)))
        operands.append(bias.reshape(1, m))
    if c_arr is not None:
        in_specs.append(pl.BlockSpec((RB, m), lambda i: (i, 0)))
        operands.append(c_arr)
    if agg_w is not None:
        agg3, w_lo, w_hi = agg_w
        kh = agg3.shape[2]
        in_specs.append(pl.BlockSpec((1, RB, kh), lambda i: (0, i, 0)))
        operands.append(agg3)
        in_specs.append(pl.BlockSpec((kh, m), lambda i: (0, 0)))
        operands.append(w_lo)
        in_specs.append(pl.BlockSpec((1, RB, kh), lambda i: (1, i, 0)))
        operands.append(agg3)
        in_specs.append(pl.BlockSpec((kh, m), lambda i: (0, 0)))
        operands.append(w_hi)
    n_out_ws = 0
    if out_ws is not None:
        mo = act_w.shape[1]
        for (a, w) in out_ws:
            k = a.shape[1]
            in_specs.append(pl.BlockSpec((RB, k), lambda i: (i, 0)))
            in_specs.append(pl.BlockSpec((k, mo), lambda i: (0, 0)))
            operands += [a, w]
        in_specs.append(pl.BlockSpec((m, mo), lambda i: (0, 0)))
        operands.append(act_w)
        in_specs.append(pl.BlockSpec((1, mo), lambda i: (0, 0)))
        operands.append(out_bias.reshape(1, mo))
        n_out_ws = len(out_ws)

    out_shapes, out_specs = [], []
    if want_pre:
        out_shapes.append(jax.ShapeDtypeStruct((NP, m), jnp.float32))
        out_specs.append(pl.BlockSpec((RB, m), lambda i: (i, 0)))
    if want_act:
        out_shapes.append(jax.ShapeDtypeStruct((NP, m), jnp.float32))
        out_specs.append(pl.BlockSpec((RB, m), lambda i: (i, 0)))
    if want_stk:
        out_shapes.append(jax.ShapeDtypeStruct((2, NP, m // 2), jnp.float32))
        out_specs.append(pl.BlockSpec((2, RB, m // 2), lambda i: (0, i, 0)))
    if out_ws is not None:
        out_shapes.append(jax.ShapeDtypeStruct((NP, act_w.shape[1]),
                                               jnp.float32))
        out_specs.append(pl.BlockSpec((RB, act_w.shape[1]),
                                      lambda i: (i, 0)))

    n_a = len(a_ws)
    has_b = bias is not None
    has_c = c_arr is not None
    has_g = agg_w is not None

    def body(*refs):
        pos = 0
        acc = None
        for j in range(n_a):
            prod = jnp.dot(refs[pos][...], refs[pos + 1][...],
                           preferred_element_type=jnp.float32)
            acc = prod if acc is None else acc + prod
            pos += 2
        if has_b:
            acc = refs[pos][...] if acc is None else acc + refs[pos][...]
            pos += 1
        if has_c:
            acc = refs[pos][...] if acc is None else acc + refs[pos][...]
            pos += 1
        if has_g:
            for _ in range(2):
                prod = jnp.dot(refs[pos][0], refs[pos + 1][...],
                               preferred_element_type=jnp.float32)
                acc = acc + prod
                pos += 2
        act = jnp.tanh(acc)
        if out_ws is not None:
            acc2 = None
            for j in range(n_out_ws):
                prod = jnp.dot(refs[pos][...], refs[pos + 1][...],
                               preferred_element_type=jnp.float32)
                acc2 = prod if acc2 is None else acc2 + prod
                pos += 2
            acc2 = acc2 + jnp.dot(act, refs[pos][...],
                                  preferred_element_type=jnp.float32)
            pos += 1
            acc2 = acc2 + refs[pos][...]
            pos += 1
        outs = refs[pos:]
        oi = 0
        if want_pre:
            outs[oi][...] = acc
            oi += 1
        if want_act:
            outs[oi][...] = act
            oi += 1
        if want_stk:
            outs[oi][0] = act[:, : m // 2]
            outs[oi][1] = act[:, m // 2:]
            oi += 1
        if out_ws is not None:
            outs[oi][...] = acc2

    res = pl.pallas_call(
        body,
        grid=grid,
        in_specs=in_specs,
        out_specs=out_specs if len(out_specs) > 1 else out_specs[0],
        out_shape=out_shapes if len(out_shapes) > 1 else out_shapes[0],
    )(*operands)
    return res if isinstance(res, (list, tuple)) else (res,)


def kernel(x, edge_index, W_s0, b_s0, W_o0, b_o0, W_s1, b_s1, W_o1, b_o1,
           W_s2, b_s2, W_o2, b_o2):
    src = edge_index[0]
    dst = edge_index[1]
    src_pad = jnp.concatenate(
        [src, jnp.zeros((EPAD - E,), jnp.int32)])
    srcs4 = jnp.stack([src_pad, src_pad + NP]).reshape(2, NS, NCHUNK, CH)
    dst3 = jnp.concatenate(
        [dst, jnp.full((EPAD - E,), N, jnp.int32)]).reshape(NS, NCHUNK, CH)
    zeros = jnp.zeros((NP, 128), jnp.float32)
    x_pad = jnp.concatenate(
        [x, jnp.zeros((NP - N, x.shape[1]), jnp.float32)])

    params = [(W_s0, b_s0, W_o0, b_o0), (W_s1, b_s1, W_o1, b_o1),
              (W_s2, b_s2, W_o2, b_o2)]
    h_comps = [x_pad]
    out = None
    for l in range(L):
        W_s, b_s, W_o, b_o = params[l]
        d = sum(hc.shape[1] for hc in h_comps)
        W_h, W_a = W_s[:d], W_s[d:]
        W_a0, W_a1 = W_a[:128], W_a[128:]
        offs = [0]
        for hc in h_comps:
            offs.append(offs[-1] + hc.shape[1])
        a_ws = [(hc, W_h[offs[j]:offs[j + 1]])
                for j, hc in enumerate(h_comps)]
        o_ws = [(hc, W_o[offs[j]:offs[j + 1]])
                for j, hc in enumerate(h_comps)]
        hWb, s_stk = _tc_mm(a_ws, b_s, None, None, True, False, True)
        for t in range(1, T):
            agg = _segsum_sc(s_stk.reshape(2 * NP, 128), srcs4, dst3, zeros)
            if t < T - 1:
                (s_stk,) = _tc_mm([], None, hWb, (agg, W_a0, W_a1),
                                  False, False, True)
            elif l < L - 1:
                s, out = _tc_mm([], None, hWb, (agg, W_a0, W_a1),
                                False, True, False,
                                out_ws=o_ws, out_bias=b_o, act_w=W_o[d:])
            else:
                (out,) = _tc_mm([], None, hWb, (agg, W_a0, W_a1),
                                False, False, False,
                                out_ws=o_ws, out_bias=b_o, act_w=W_o[d:])
        if l < L - 1:
            h_comps = h_comps + [s, out]
    return out[:N]
